# Initial kernel scaffold; baseline (speedup 1.0000x reference)
import jax
import jax.numpy as jnp
from jax.experimental import pallas as pl


def kernel(inputs, table):
    raise NotImplementedError("write your pallas kernel here")


# speedup vs baseline: 2.5179x; 2.5179x over previous
import jax
import jax.numpy as jnp
from jax.experimental import pallas as pl

MAXLEN = 8192
OUTPUT_DIM = 2048
ROWS_PER_BLOCK = 512


def _copy_block(table_ref, out_ref):
    out_ref[...] = table_ref[...]


def kernel(inputs, table):
    del inputs
    out = pl.pallas_call(
        _copy_block,
        grid=(MAXLEN // ROWS_PER_BLOCK,),
        in_specs=[pl.BlockSpec((ROWS_PER_BLOCK, OUTPUT_DIM), lambda i: (i, 0))],
        out_specs=pl.BlockSpec((ROWS_PER_BLOCK, OUTPUT_DIM), lambda i: (i, 0)),
        out_shape=jax.ShapeDtypeStruct((MAXLEN, OUTPUT_DIM), table.dtype),
    )(table)
    return out[None]
